# baseline (speedup 1.0000x reference)
import functools

import jax
import jax.numpy as jnp
from jax import lax
from jax.experimental import pallas as pl
from jax.experimental.pallas import tpu as pltpu
from jax.experimental.pallas import tpu_sc as plsc

B = 16384
NW = 21
NUM_CORES = 2
NUM_SUBCORES = 16
L = 16
NWORK = NUM_CORES * NUM_SUBCORES
ROWS = B // NWORK
GROUPS = ROWS // L


@functools.partial(
    pl.kernel,
    mesh=plsc.VectorSubcoreMesh(core_axis_name="c", subcore_axis_name="s"),
    compiler_params=pltpu.CompilerParams(needs_layout_passes=False),
    out_type=[
        jax.ShapeDtypeStruct((B,), jnp.float32),
        jax.ShapeDtypeStruct((B,), jnp.float32),
        jax.ShapeDtypeStruct((NW, B), jnp.float32),
        jax.ShapeDtypeStruct((B,), jnp.int32),
    ],
    scratch_types=[
        pltpu.VMEM((NW, ROWS), jnp.float32),
        pltpu.VMEM((NW * 2, ROWS), jnp.float32),
        pltpu.VMEM((NW, ROWS), jnp.float32),
        pltpu.VMEM((ROWS,), jnp.float32),
        pltpu.VMEM((ROWS,), jnp.float32),
        pltpu.VMEM((ROWS,), jnp.int32),
        pltpu.SemaphoreType.DMA,
        pltpu.SemaphoreType.DMA,
        pltpu.SemaphoreType.DMA,
        pltpu.SemaphoreType.DMA,
    ],
)
def _sc_extract(qT_hbm, pT_hbm, high_hbm, low_hbm, validT_hbm, idx_hbm,
                q_v, p_v, valid_v, high_v, low_v, idx_v,
                sem_q, sem_p, sem_vi, sem_hl):
    wid = lax.axis_index("s") * NUM_CORES + lax.axis_index("c")
    row0 = wid * ROWS
    cq = pltpu.async_copy(qT_hbm.at[:, pl.ds(row0, ROWS)], q_v, sem_q)
    cp = pltpu.async_copy(pT_hbm.at[:, pl.ds(row0, ROWS)], p_v, sem_p)

    zeros = jnp.zeros((L,), jnp.float32)
    ones = jnp.ones((L,), jnp.float32)
    for w in range(NW):
        for g in range(GROUPS):
            valid_v[w, pl.ds(g * L, L)] = zeros
    cq.wait()

    lanes = lax.iota(jnp.int32, L)
    best_ws = []
    for g in range(GROUPS):
        col = g * L
        best_v = q_v[0, pl.ds(col, L)]
        best_w = jnp.zeros((L,), jnp.int32)
        for w in range(1, NW):
            v = q_v[w, pl.ds(col, L)]
            gt = v > best_v
            best_v = jnp.where(gt, v, best_v)
            best_w = jnp.where(gt, w, best_w)
        plsc.store_scatter(valid_v, [best_w, lanes + col], ones)
        idx_v[pl.ds(col, L)] = best_w
        best_ws.append(best_w)

    cv = pltpu.async_copy(valid_v, validT_hbm.at[:, pl.ds(row0, ROWS)], sem_vi)
    ci = pltpu.async_copy(idx_v, idx_hbm.at[pl.ds(row0, ROWS)], sem_vi)
    cp.wait()

    for g in range(GROUPS):
        col = g * L
        cols = lanes + col
        best_w = best_ws[g]
        high_v[pl.ds(col, L)] = plsc.load_gather(p_v, [best_w * 2, cols])
        low_v[pl.ds(col, L)] = plsc.load_gather(p_v, [best_w * 2 + 1, cols])

    ch = pltpu.async_copy(high_v, high_hbm.at[pl.ds(row0, ROWS)], sem_hl)
    cl = pltpu.async_copy(low_v, low_hbm.at[pl.ds(row0, ROWS)], sem_hl)
    cv.wait()
    ci.wait()
    ch.wait()
    cl.wait()


def kernel(hidden_state, projections, quality_scores, r_squared,
           complete_cycles, position):
    del hidden_state, r_squared, complete_cycles, position
    qT = quality_scores.T
    pT = projections.transpose(1, 2, 0).reshape(NW * 2, B)
    high, low, validT, idx = _sc_extract(qT, pT)
    return (high[:, None], low[:, None], validT.T, idx)


# speedup vs baseline: 1.1321x; 1.1321x over previous
import functools

import jax
import jax.numpy as jnp
from jax import lax
from jax.experimental import pallas as pl
from jax.experimental.pallas import tpu as pltpu
from jax.experimental.pallas import tpu_sc as plsc

B = 16384
NW = 21
NUM_CORES = 2
NUM_SUBCORES = 16
L = 16
NWORK = NUM_CORES * NUM_SUBCORES
ROWS = B // NWORK
GROUPS = ROWS // L


@functools.partial(
    pl.kernel,
    mesh=plsc.VectorSubcoreMesh(core_axis_name="c", subcore_axis_name="s"),
    compiler_params=pltpu.CompilerParams(needs_layout_passes=False),
    out_type=[
        jax.ShapeDtypeStruct((B,), jnp.float32),
        jax.ShapeDtypeStruct((B,), jnp.float32),
        jax.ShapeDtypeStruct((NW, B), jnp.float32),
        jax.ShapeDtypeStruct((B,), jnp.int32),
    ],
    scratch_types=[
        pltpu.VMEM((NW, ROWS), jnp.float32),
        pltpu.VMEM((NW * 2, ROWS), jnp.float32),
        pltpu.VMEM((NW, ROWS), jnp.float32),
        pltpu.VMEM((ROWS,), jnp.float32),
        pltpu.VMEM((ROWS,), jnp.float32),
        pltpu.VMEM((ROWS,), jnp.int32),
        pltpu.SemaphoreType.DMA,
        pltpu.SemaphoreType.DMA,
        pltpu.SemaphoreType.DMA,
        pltpu.SemaphoreType.DMA,
    ],
)
def _sc_extract(qT_hbm, pT_hbm, high_hbm, low_hbm, validT_hbm, idx_hbm,
                q_v, p_v, valid_v, high_v, low_v, idx_v,
                sem_q, sem_p, sem_vi, sem_hl):
    wid = lax.axis_index("s") * NUM_CORES + lax.axis_index("c")
    row0 = wid * ROWS
    cq = pltpu.async_copy(qT_hbm.at[:, pl.ds(row0, ROWS)], q_v, sem_q)
    cp = pltpu.async_copy(pT_hbm.at[:, pl.ds(row0, ROWS)], p_v, sem_p)

    cq.wait()

    lanes = lax.iota(jnp.int32, L)
    best_ws = []
    for g in range(GROUPS):
        col = g * L
        best_v = q_v[0, pl.ds(col, L)]
        best_w = jnp.zeros((L,), jnp.int32)
        for w in range(1, NW):
            v = q_v[w, pl.ds(col, L)]
            gt = v > best_v
            best_v = jnp.where(gt, v, best_v)
            best_w = jnp.where(gt, w, best_w)
        for w in range(NW):
            valid_v[w, pl.ds(col, L)] = jnp.where(
                best_w == w, 1.0, 0.0).astype(jnp.float32)
        idx_v[pl.ds(col, L)] = best_w
        best_ws.append(best_w)

    cv = pltpu.async_copy(valid_v, validT_hbm.at[:, pl.ds(row0, ROWS)], sem_vi)
    ci = pltpu.async_copy(idx_v, idx_hbm.at[pl.ds(row0, ROWS)], sem_vi)
    cp.wait()

    for g in range(GROUPS):
        col = g * L
        cols = lanes + col
        best_w = best_ws[g]
        high_v[pl.ds(col, L)] = plsc.load_gather(p_v, [best_w * 2, cols])
        low_v[pl.ds(col, L)] = plsc.load_gather(p_v, [best_w * 2 + 1, cols])

    ch = pltpu.async_copy(high_v, high_hbm.at[pl.ds(row0, ROWS)], sem_hl)
    cl = pltpu.async_copy(low_v, low_hbm.at[pl.ds(row0, ROWS)], sem_hl)
    cv.wait()
    ci.wait()
    ch.wait()
    cl.wait()


def kernel(hidden_state, projections, quality_scores, r_squared,
           complete_cycles, position):
    del hidden_state, r_squared, complete_cycles, position
    qT = quality_scores.T
    pT = projections.transpose(1, 2, 0).reshape(NW * 2, B)
    high, low, validT, idx = _sc_extract(qT, pT)
    return (high[:, None], low[:, None], validT.T, idx)
